# Initial kernel scaffold; baseline (speedup 1.0000x reference)
import jax
import jax.numpy as jnp
from jax.experimental import pallas as pl


def kernel(indices, table):
    raise NotImplementedError("write your pallas kernel here")


# speedup vs baseline: 2.8241x; 2.8241x over previous
import functools

import jax
import jax.numpy as jnp
from jax import lax
from jax.experimental import pallas as pl
from jax.experimental.pallas import tpu as pltpu
from jax.experimental.pallas import tpu_sc as plsc

VOCAB = 100000
DIM = 128
BATCH = 16384
HIST = 50
TOT = BATCH * HIST

_info = plsc.get_sparse_core_info()
NC, NS = _info.num_cores, _info.num_subcores
NW = NC * NS
PER_W = TOT // NW
CHUNK = 128
NCH = PER_W // CHUNK


def _make_gather():
    mesh = plsc.VectorSubcoreMesh(core_axis_name="c", subcore_axis_name="s")

    @functools.partial(
        pl.kernel,
        mesh=mesh,
        out_type=jax.ShapeDtypeStruct((TOT, DIM), jnp.float32),
        scratch_types=[
            pltpu.VMEM((CHUNK,), jnp.int32),
            pltpu.VMEM((CHUNK, DIM), jnp.float32),
            pltpu.SemaphoreType.DMA,
        ],
    )
    def gather_kernel(idx_hbm, table_hbm, out_hbm, idx_v, rows_v, gsem):
        wid = lax.axis_index("s") * NC + lax.axis_index("c")
        base = wid * PER_W

        def chunk_body(g, carry):
            off = base + g * CHUNK
            pltpu.sync_copy(idx_hbm.at[pl.ds(off, CHUNK)], idx_v)
            pltpu.async_copy(table_hbm.at[idx_v], rows_v, gsem).wait()
            pltpu.sync_copy(rows_v, out_hbm.at[pl.ds(off, CHUNK)])
            return carry

        lax.fori_loop(0, NCH, chunk_body, 0)

    return gather_kernel


_gather = _make_gather()


def kernel(indices, table):
    flat = jnp.reshape(indices, (TOT,)).astype(jnp.int32)
    out = _gather(flat, table)
    return jnp.reshape(out, (BATCH, HIST, DIM))


# speedup vs baseline: 3.4749x; 1.2304x over previous
import functools

import jax
import jax.numpy as jnp
from jax import lax
from jax.experimental import pallas as pl
from jax.experimental.pallas import tpu as pltpu
from jax.experimental.pallas import tpu_sc as plsc

VOCAB = 100000
DIM = 128
BATCH = 16384
HIST = 50
TOT = BATCH * HIST

_info = plsc.get_sparse_core_info()
NC, NS = _info.num_cores, _info.num_subcores
NW = NC * NS
PER_W = TOT // NW
CHUNK = 128
NCH = PER_W // CHUNK
NB = 4
NSTEP = NCH // NB


def _make_gather():
    mesh = plsc.VectorSubcoreMesh(core_axis_name="c", subcore_axis_name="s")

    @functools.partial(
        pl.kernel,
        mesh=mesh,
        out_type=jax.ShapeDtypeStruct((TOT, DIM), jnp.float32),
        scratch_types=(
            [pltpu.VMEM((PER_W,), jnp.int32)]
            + [pltpu.VMEM((CHUNK, DIM), jnp.float32) for _ in range(NB)]
            + [pltpu.SemaphoreType.DMA for _ in range(2 * NB)]
        ),
    )
    def gather_kernel(idx_hbm, table_hbm, out_hbm, idx_v, *bufs_and_sems):
        rows = bufs_and_sems[:NB]
        gsem = bufs_and_sems[NB : 2 * NB]
        wsem = bufs_and_sems[2 * NB : 3 * NB]

        wid = lax.axis_index("s") * NC + lax.axis_index("c")
        base = wid * PER_W

        pltpu.sync_copy(idx_hbm.at[pl.ds(base, PER_W)], idx_v)

        def start_gather(t, b):
            idx_slice = idx_v.at[pl.ds(t * CHUNK, CHUNK)]
            pltpu.async_copy(table_hbm.at[idx_slice], rows[b], gsem[b])

        def start_store(t, b):
            pltpu.async_copy(rows[b], out_hbm.at[pl.ds(base + t * CHUNK, CHUNK)], wsem[b])

        def wait_store(b):
            pltpu.make_async_copy(
                rows[b], out_hbm.at[pl.ds(base, CHUNK)], wsem[b]
            ).wait()

        def wait_gather(b):
            pltpu.make_async_copy(
                table_hbm.at[idx_v.at[pl.ds(0, CHUNK)]], rows[b], gsem[b]
            ).wait()

        start_gather(0, 0)
        start_gather(1, 1)

        def step_body(s, carry):
            for b in range(NB):
                t = s * NB + b
                gn = t + 2
                bg = (b + 2) % NB

                @pl.when(jnp.logical_and(gn >= NB, gn < NCH))
                def _():
                    wait_store(bg)

                @pl.when(gn < NCH)
                def _():
                    start_gather(gn, bg)

                wait_gather(b)
                start_store(t, b)
            return carry

        lax.fori_loop(0, NSTEP, step_body, 0)

        for b in range(NB):
            wait_store(b)

    return gather_kernel


_gather = _make_gather()


def kernel(indices, table):
    flat = jnp.reshape(indices, (TOT,)).astype(jnp.int32)
    out = _gather(flat, table)
    return jnp.reshape(out, (BATCH, HIST, DIM))
